# baseline (device time: 1242707 ns/iter reference)
import jax
import jax.numpy as jnp
from jax import lax
from jax.experimental import pallas as pl
from jax.experimental.pallas import tpu as pltpu

N_DEV = 4
NT = 1024


def kernel(x, w_mat):
    m_total, k_loc = x.shape
    k_loc2, n_total = w_mat.shape
    assert k_loc == k_loc2
    m_blk = m_total // N_DEV
    num_tiles = n_total // NT

    def body(x_ref, w_ref, out_ref, comm_ref, send_sems, recv_sems, credit_sem):
        t = pl.program_id(0)
        my = lax.axis_index("i")
        left = (my + N_DEV - 1) % N_DEV
        right = (my + 1) % N_DEV

        @pl.when(t == 0)
        def _():
            barrier_sem = pltpu.get_barrier_semaphore()
            for nbr in (left, right):
                pl.semaphore_signal(
                    barrier_sem, inc=1,
                    device_id=(nbr,), device_id_type=pl.DeviceIdType.MESH,
                )
            pl.semaphore_wait(barrier_sem, 2)

        @pl.when(t > 0)
        def _():
            pl.semaphore_wait(credit_sem, 1)

        def partial(j):
            xb = x_ref[pl.ds(j * m_blk, m_blk), :]
            return jnp.dot(xb, w_ref[:, :], preferred_element_type=jnp.float32)

        comm_ref[3, :, :] = partial((my + N_DEV - 1) % N_DEV)

        src_slots = (3, 0, 1)
        for h in range(N_DEV - 1):
            rdma = pltpu.make_async_remote_copy(
                src_ref=comm_ref.at[src_slots[h]],
                dst_ref=comm_ref.at[h],
                send_sem=send_sems.at[h],
                recv_sem=recv_sems.at[h],
                device_id=(right,),
                device_id_type=pl.DeviceIdType.MESH,
            )
            rdma.start()
            rdma.wait()
            if h < N_DEV - 2:
                j = (my + (N_DEV - 2 - h)) % N_DEV
                comm_ref[h, :, :] = comm_ref[h, :, :] + partial(j)

        out_ref[:, :] = comm_ref[N_DEV - 2, :, :] + partial(my)

        pl.semaphore_signal(
            credit_sem, inc=1,
            device_id=(left,), device_id_type=pl.DeviceIdType.MESH,
        )

        @pl.when(t == num_tiles - 1)
        def _():
            pl.semaphore_wait(credit_sem, 1)

    return pl.pallas_call(
        body,
        grid=(num_tiles,),
        out_shape=jax.ShapeDtypeStruct((m_blk, n_total), jnp.float32),
        in_specs=[
            pl.BlockSpec(memory_space=pltpu.VMEM),
            pl.BlockSpec((k_loc, NT), lambda t: (0, t)),
        ],
        out_specs=pl.BlockSpec((m_blk, NT), lambda t: (0, t)),
        scratch_shapes=[
            pltpu.VMEM((4, m_blk, NT), jnp.float32),
            pltpu.SemaphoreType.DMA((N_DEV - 1,)),
            pltpu.SemaphoreType.DMA((N_DEV - 1,)),
            pltpu.SemaphoreType.REGULAR,
        ],
        compiler_params=pltpu.CompilerParams(collective_id=0),
    )(x, w_mat)


# device time: 648453 ns/iter; 1.9164x vs baseline; 1.9164x over previous
import jax
import jax.numpy as jnp
from jax import lax
from jax.experimental import pallas as pl
from jax.experimental.pallas import tpu as pltpu

N_DEV = 4
NT = 1024
H = NT // 2


def kernel(x, w_mat):
    m_total, k_loc = x.shape
    k_loc2, n_total = w_mat.shape
    assert k_loc == k_loc2
    m_blk = m_total // N_DEV
    num_tiles = n_total // NT

    def body(x_ref, w_ref, out_ref, comm_r, comm_l,
             send_r, recv_r, send_l, recv_l, credit_r, credit_l):
        t = pl.program_id(0)
        my = lax.axis_index("i")
        left = (my + N_DEV - 1) % N_DEV
        right = (my + 1) % N_DEV

        @pl.when(t == 0)
        def _():
            barrier_sem = pltpu.get_barrier_semaphore()
            for nbr in (left, right):
                pl.semaphore_signal(
                    barrier_sem, inc=1,
                    device_id=(nbr,), device_id_type=pl.DeviceIdType.MESH,
                )
            pl.semaphore_wait(barrier_sem, 2)

        @pl.when(t > 0)
        def _():
            pl.semaphore_wait(credit_r, 1)
            pl.semaphore_wait(credit_l, 1)

        def pR(j):
            xb = x_ref[pl.ds(j * m_blk, m_blk), :]
            return jnp.dot(xb, w_ref[:, :H], preferred_element_type=jnp.float32)

        def pL(j):
            xb = x_ref[pl.ds(j * m_blk, m_blk), :]
            return jnp.dot(xb, w_ref[:, H:], preferred_element_type=jnp.float32)

        def mk(buf, src_slot, dst_slot, ssems, rsems, h, dev):
            return pltpu.make_async_remote_copy(
                src_ref=buf.at[src_slot],
                dst_ref=buf.at[dst_slot],
                send_sem=ssems.at[h],
                recv_sem=rsems.at[h],
                device_id=(dev,),
                device_id_type=pl.DeviceIdType.MESH,
            )

        comm_r[3, :, :] = pR((my + N_DEV - 1) % N_DEV)
        r_rdmas = [mk(comm_r, 3, 0, send_r, recv_r, 0, right)]
        r_rdmas[0].start()
        comm_l[3, :, :] = pL((my + 1) % N_DEV)
        l_rdmas = [mk(comm_l, 3, 0, send_l, recv_l, 0, left)]
        l_rdmas[0].start()

        for h in range(N_DEV - 2):
            prv = pR((my + 2 - h) % N_DEV)
            plv = pL((my + 2 + h) % N_DEV)
            r_rdmas[h].wait_recv()
            comm_r[h, :, :] = comm_r[h, :, :] + prv
            nxt = mk(comm_r, h, h + 1, send_r, recv_r, h + 1, right)
            nxt.start()
            r_rdmas.append(nxt)
            l_rdmas[h].wait_recv()
            comm_l[h, :, :] = comm_l[h, :, :] + plv
            nxt = mk(comm_l, h, h + 1, send_l, recv_l, h + 1, left)
            nxt.start()
            l_rdmas.append(nxt)

        pr_f = pR(my)
        pl_f = pL(my)
        r_rdmas[2].wait_recv()
        out_ref[:, :H] = comm_r[2, :, :] + pr_f
        l_rdmas[2].wait_recv()
        out_ref[:, H:] = comm_l[2, :, :] + pl_f

        for r in r_rdmas:
            r.wait_send()
        for r in l_rdmas:
            r.wait_send()

        pl.semaphore_signal(
            credit_r, inc=1,
            device_id=(left,), device_id_type=pl.DeviceIdType.MESH,
        )
        pl.semaphore_signal(
            credit_l, inc=1,
            device_id=(right,), device_id_type=pl.DeviceIdType.MESH,
        )

        @pl.when(t == num_tiles - 1)
        def _():
            pl.semaphore_wait(credit_r, 1)
            pl.semaphore_wait(credit_l, 1)

    return pl.pallas_call(
        body,
        grid=(num_tiles,),
        out_shape=jax.ShapeDtypeStruct((m_blk, n_total), jnp.float32),
        in_specs=[
            pl.BlockSpec(memory_space=pltpu.VMEM),
            pl.BlockSpec((k_loc, NT), lambda t: (0, t)),
        ],
        out_specs=pl.BlockSpec((m_blk, NT), lambda t: (0, t)),
        scratch_shapes=[
            pltpu.VMEM((4, m_blk, H), jnp.float32),
            pltpu.VMEM((4, m_blk, H), jnp.float32),
            pltpu.SemaphoreType.DMA((N_DEV - 1,)),
            pltpu.SemaphoreType.DMA((N_DEV - 1,)),
            pltpu.SemaphoreType.DMA((N_DEV - 1,)),
            pltpu.SemaphoreType.DMA((N_DEV - 1,)),
            pltpu.SemaphoreType.REGULAR,
            pltpu.SemaphoreType.REGULAR,
        ],
        compiler_params=pltpu.CompilerParams(
            collective_id=0,
            vmem_limit_bytes=64 * 1024 * 1024,
        ),
    )(x, w_mat)


# device time: 380525 ns/iter; 3.2658x vs baseline; 1.7041x over previous
import jax
import jax.numpy as jnp
from jax import lax
from jax.experimental import pallas as pl
from jax.experimental.pallas import tpu as pltpu

N_DEV = 4
NT = 1024
H = NT // 2


def kernel(x, w_mat):
    m_total, k_loc = x.shape
    k_loc2, n_total = w_mat.shape
    assert k_loc == k_loc2
    m_blk = m_total // N_DEV
    num_tiles = n_total // NT

    def body(x_ref, w_ref, out_ref, comm_r, comm_l,
             send_r, recv_r, send_l, recv_l, credit_r, credit_l):
        t = pl.program_id(0)
        my = lax.axis_index("i")
        left = (my + N_DEV - 1) % N_DEV
        right = (my + 1) % N_DEV

        @pl.when(t == 0)
        def _():
            barrier_sem = pltpu.get_barrier_semaphore()
            for nbr in (left, right):
                pl.semaphore_signal(
                    barrier_sem, inc=1,
                    device_id=(nbr,), device_id_type=pl.DeviceIdType.MESH,
                )
            pl.semaphore_wait(barrier_sem, 2)

        @pl.when(t > 0)
        def _():
            pl.semaphore_wait(credit_r, 1)
            pl.semaphore_wait(credit_l, 1)

        def pR(j):
            xb = x_ref[pl.ds(j * m_blk, m_blk), :]
            return jnp.dot(xb, w_ref[:, :H], preferred_element_type=jnp.float32)

        def pL(j):
            xb = x_ref[pl.ds(j * m_blk, m_blk), :]
            return jnp.dot(xb, w_ref[:, H:], preferred_element_type=jnp.float32)

        def mk(buf, src_slot, dst_slot, ssems, rsems, h, dev):
            return pltpu.make_async_remote_copy(
                src_ref=buf.at[src_slot],
                dst_ref=buf.at[dst_slot],
                send_sem=ssems.at[h],
                recv_sem=rsems.at[h],
                device_id=(dev,),
                device_id_type=pl.DeviceIdType.MESH,
            )

        comm_r[3, :, :] = pR((my + N_DEV - 1) % N_DEV).astype(jnp.bfloat16)
        r_rdmas = [mk(comm_r, 3, 0, send_r, recv_r, 0, right)]
        r_rdmas[0].start()
        comm_l[3, :, :] = pL((my + 1) % N_DEV).astype(jnp.bfloat16)
        l_rdmas = [mk(comm_l, 3, 0, send_l, recv_l, 0, left)]
        l_rdmas[0].start()

        for h in range(N_DEV - 2):
            prv = pR((my + 2 - h) % N_DEV)
            plv = pL((my + 2 + h) % N_DEV)
            r_rdmas[h].wait_recv()
            comm_r[h, :, :] = (comm_r[h, :, :].astype(jnp.float32) + prv).astype(jnp.bfloat16)
            nxt = mk(comm_r, h, h + 1, send_r, recv_r, h + 1, right)
            nxt.start()
            r_rdmas.append(nxt)
            l_rdmas[h].wait_recv()
            comm_l[h, :, :] = (comm_l[h, :, :].astype(jnp.float32) + plv).astype(jnp.bfloat16)
            nxt = mk(comm_l, h, h + 1, send_l, recv_l, h + 1, left)
            nxt.start()
            l_rdmas.append(nxt)

        pr_f = pR(my)
        pl_f = pL(my)
        r_rdmas[2].wait_recv()
        out_ref[:, :H] = comm_r[2, :, :].astype(jnp.float32) + pr_f
        l_rdmas[2].wait_recv()
        out_ref[:, H:] = comm_l[2, :, :].astype(jnp.float32) + pl_f

        for r in r_rdmas:
            r.wait_send()
        for r in l_rdmas:
            r.wait_send()

        pl.semaphore_signal(
            credit_r, inc=1,
            device_id=(left,), device_id_type=pl.DeviceIdType.MESH,
        )
        pl.semaphore_signal(
            credit_l, inc=1,
            device_id=(right,), device_id_type=pl.DeviceIdType.MESH,
        )

        @pl.when(t == num_tiles - 1)
        def _():
            pl.semaphore_wait(credit_r, 1)
            pl.semaphore_wait(credit_l, 1)

    return pl.pallas_call(
        body,
        grid=(num_tiles,),
        out_shape=jax.ShapeDtypeStruct((m_blk, n_total), jnp.float32),
        in_specs=[
            pl.BlockSpec(memory_space=pltpu.VMEM),
            pl.BlockSpec((k_loc, NT), lambda t: (0, t)),
        ],
        out_specs=pl.BlockSpec((m_blk, NT), lambda t: (0, t)),
        scratch_shapes=[
            pltpu.VMEM((4, m_blk, H), jnp.bfloat16),
            pltpu.VMEM((4, m_blk, H), jnp.bfloat16),
            pltpu.SemaphoreType.DMA((N_DEV - 1,)),
            pltpu.SemaphoreType.DMA((N_DEV - 1,)),
            pltpu.SemaphoreType.DMA((N_DEV - 1,)),
            pltpu.SemaphoreType.DMA((N_DEV - 1,)),
            pltpu.SemaphoreType.REGULAR,
            pltpu.SemaphoreType.REGULAR,
        ],
        compiler_params=pltpu.CompilerParams(
            collective_id=0,
            vmem_limit_bytes=64 * 1024 * 1024,
        ),
    )(x, w_mat)


# device time: 311027 ns/iter; 3.9955x vs baseline; 1.2234x over previous
import jax
import jax.numpy as jnp
from jax import lax
from jax.experimental import pallas as pl
from jax.experimental.pallas import tpu as pltpu

N_DEV = 4
NT = 512
H = NT // 2


def kernel(x, w_mat):
    m_total, k_loc = x.shape
    k_loc2, n_total = w_mat.shape
    assert k_loc == k_loc2
    m_blk = m_total // N_DEV
    T = n_total // NT
    S = T + 3

    def body(x_ref, w0_ref, w1_ref, w2_ref, w3_ref, out_ref,
             comm_r, comm_l, send_r, recv_r, send_l, recv_l,
             credit_r, credit_l):
        s = pl.program_id(0)
        my = lax.axis_index("i")
        left = (my + N_DEV - 1) % N_DEV
        right = (my + 1) % N_DEV

        @pl.when(s == 0)
        def _():
            barrier_sem = pltpu.get_barrier_semaphore()
            for nbr in (left, right):
                pl.semaphore_signal(
                    barrier_sem, inc=1,
                    device_id=(nbr,), device_id_type=pl.DeviceIdType.MESH,
                )
            pl.semaphore_wait(barrier_sem, 2)

        @pl.when(s >= 1)
        def _():
            pl.semaphore_wait(credit_r, 1)
            pl.semaphore_wait(credit_l, 1)

        sp = [(s + 4 - k) % 4 for k in range(4)]
        pp = [(s + 2 - k) % 2 for k in range(4)]

        def part(j, w_ref, lo):
            xb = x_ref[pl.ds(j * m_blk, m_blk), :]
            return jnp.dot(
                xb, w_ref[:, lo:lo + H], preferred_element_type=jnp.float32
            )

        dirs = (
            (comm_r, send_r, recv_r, right, 0,
             ((my + 3) % N_DEV, (my + 2) % N_DEV, (my + 1) % N_DEV, my)),
            (comm_l, send_l, recv_l, left, H,
             ((my + 1) % N_DEV, (my + 2) % N_DEV, (my + 3) % N_DEV, my)),
        )

        def fwd(comm, ssems, rsems, dev, src, dst, spar, ppar, h):
            return pltpu.make_async_remote_copy(
                src_ref=comm.at[spar, src],
                dst_ref=comm.at[spar, dst],
                send_sem=ssems.at[ppar, h],
                recv_sem=rsems.at[ppar, h],
                device_id=(dev,),
                device_id_type=pl.DeviceIdType.MESH,
            )

        @pl.when(s < T)
        def _():
            for comm, ssems, rsems, dev, lo, chunks in dirs:
                d = fwd(comm, ssems, rsems, dev, 3, 0, sp[0], pp[0], 0)

                @pl.when(s >= 2)
                def _():
                    d.wait_send()
                comm[sp[0], 3, :, :] = part(chunks[0], w0_ref, lo).astype(
                    jnp.bfloat16
                )
                d.start()

        @pl.when(jnp.logical_and(s >= 1, s <= T))
        def _():
            for comm, ssems, rsems, dev, lo, chunks in dirs:
                prv = part(chunks[1], w1_ref, lo)
                rec = fwd(comm, ssems, rsems, dev, 0, 0, sp[1], pp[1], 0)
                rec.wait_recv()
                snd = fwd(comm, ssems, rsems, dev, 0, 1, sp[1], pp[1], 1)

                @pl.when(s >= 3)
                def _():
                    snd.wait_send()
                comm[sp[1], 0, :, :] = (
                    comm[sp[1], 0, :, :].astype(jnp.float32) + prv
                ).astype(jnp.bfloat16)
                snd.start()

        @pl.when(jnp.logical_and(s >= 2, s <= T + 1))
        def _():
            for comm, ssems, rsems, dev, lo, chunks in dirs:
                prv = part(chunks[2], w2_ref, lo)
                rec = fwd(comm, ssems, rsems, dev, 1, 1, sp[2], pp[2], 1)
                rec.wait_recv()
                snd = fwd(comm, ssems, rsems, dev, 1, 2, sp[2], pp[2], 2)

                @pl.when(s >= 4)
                def _():
                    snd.wait_send()
                comm[sp[2], 1, :, :] = (
                    comm[sp[2], 1, :, :].astype(jnp.float32) + prv
                ).astype(jnp.bfloat16)
                snd.start()

        @pl.when(s >= 3)
        def _():
            for comm, ssems, rsems, dev, lo, chunks in dirs:
                prv = part(chunks[3], w3_ref, lo)
                rec = fwd(comm, ssems, rsems, dev, 2, 2, sp[3], pp[3], 2)
                rec.wait_recv()
                out_ref[:, lo:lo + H] = (
                    comm[sp[3], 2, :, :].astype(jnp.float32) + prv
                )

        pl.semaphore_signal(
            credit_r, inc=1,
            device_id=(left,), device_id_type=pl.DeviceIdType.MESH,
        )
        pl.semaphore_signal(
            credit_l, inc=1,
            device_id=(right,), device_id_type=pl.DeviceIdType.MESH,
        )

        @pl.when(s == S - 1)
        def _():
            for comm, ssems, rsems, dev, lo, chunks in dirs:
                for par in (0, 1):
                    for h in range(3):
                        fwd(comm, ssems, rsems, dev, 3, 0, 0, par, h).wait_send()
            pl.semaphore_wait(credit_r, 1)
            pl.semaphore_wait(credit_l, 1)

    clamp = lambda v: jnp.clip(v, 0, T - 1)
    return pl.pallas_call(
        body,
        grid=(S,),
        out_shape=jax.ShapeDtypeStruct((m_blk, n_total), jnp.float32),
        in_specs=[
            pl.BlockSpec(memory_space=pltpu.VMEM),
            pl.BlockSpec((k_loc, NT), lambda s: (0, clamp(s))),
            pl.BlockSpec((k_loc, NT), lambda s: (0, clamp(s - 1))),
            pl.BlockSpec((k_loc, NT), lambda s: (0, clamp(s - 2))),
            pl.BlockSpec((k_loc, NT), lambda s: (0, clamp(s - 3))),
        ],
        out_specs=pl.BlockSpec((m_blk, NT), lambda s: (0, clamp(s - 3))),
        scratch_shapes=[
            pltpu.VMEM((4, 4, m_blk, H), jnp.bfloat16),
            pltpu.VMEM((4, 4, m_blk, H), jnp.bfloat16),
            pltpu.SemaphoreType.DMA((2, 3)),
            pltpu.SemaphoreType.DMA((2, 3)),
            pltpu.SemaphoreType.DMA((2, 3)),
            pltpu.SemaphoreType.DMA((2, 3)),
            pltpu.SemaphoreType.REGULAR,
            pltpu.SemaphoreType.REGULAR,
        ],
        compiler_params=pltpu.CompilerParams(
            collective_id=0,
            vmem_limit_bytes=64 * 1024 * 1024,
        ),
    )(x, w_mat, w_mat, w_mat, w_mat)
